# Initial kernel scaffold; baseline (speedup 1.0000x reference)
import jax
import jax.numpy as jnp
from jax.experimental import pallas as pl


def kernel(x, table):
    raise NotImplementedError("write your pallas kernel here")


# speedup vs baseline: 6.3430x; 6.3430x over previous
import functools

import jax
import jax.numpy as jnp
from jax import lax
from jax.experimental import pallas as pl
from jax.experimental.pallas import tpu as pltpu
from jax.experimental.pallas import tpu_sc as plsc

VOCAB = 100000
EMBED_DIM = 128
NUM_CORES = 2
NUM_SUBCORES = 16
NUM_WORKERS = NUM_CORES * NUM_SUBCORES
CHUNK = 128


def _embed_kernel(total, chunks_per_worker):
    mesh = plsc.VectorSubcoreMesh(core_axis_name="c", subcore_axis_name="s")

    @functools.partial(
        pl.kernel,
        out_type=jax.ShapeDtypeStruct((total, EMBED_DIM), jnp.float32),
        mesh=mesh,
        scratch_types=[
            pltpu.VMEM((chunks_per_worker, CHUNK), jnp.int32),
            pltpu.VMEM((CHUNK, EMBED_DIM), jnp.float32),
            pltpu.SemaphoreType.DMA,
        ],
    )
    def k(idx_hbm, table_hbm, out_hbm, idx_v, rows_v, gsem):
        wid = lax.axis_index("s") * NUM_CORES + lax.axis_index("c")
        base = wid * (chunks_per_worker * CHUNK)
        pltpu.sync_copy(idx_hbm.at[wid], idx_v)

        @pl.loop(0, chunks_per_worker)
        def chunk_loop(j):
            pltpu.async_copy(table_hbm.at[idx_v.at[j]], rows_v, gsem).wait()
            pltpu.sync_copy(rows_v, out_hbm.at[pl.ds(base + j * CHUNK, CHUNK)])

    return k


def kernel(x, table):
    batch, hist = x.shape
    total = batch * hist
    chunks_per_worker = total // (NUM_WORKERS * CHUNK)
    idx = x.astype(jnp.int32).reshape(NUM_WORKERS, chunks_per_worker, CHUNK)
    out = _embed_kernel(total, chunks_per_worker)(idx, table)
    return out.reshape(batch, hist, EMBED_DIM)


# speedup vs baseline: 7.5234x; 1.1861x over previous
import functools

import jax
import jax.numpy as jnp
from jax import lax
from jax.experimental import pallas as pl
from jax.experimental.pallas import tpu as pltpu
from jax.experimental.pallas import tpu_sc as plsc

VOCAB = 100000
EMBED_DIM = 128
NUM_CORES = 2
NUM_SUBCORES = 16
NUM_WORKERS = NUM_CORES * NUM_SUBCORES
CHUNK = 128


def _embed_kernel(total, chunks_per_worker):
    mesh = plsc.VectorSubcoreMesh(core_axis_name="c", subcore_axis_name="s")

    n = chunks_per_worker

    @functools.partial(
        pl.kernel,
        out_type=jax.ShapeDtypeStruct((total, EMBED_DIM), jnp.float32),
        mesh=mesh,
        scratch_types=[
            pltpu.VMEM((n, CHUNK), jnp.int32),
            pltpu.VMEM((2, CHUNK, EMBED_DIM), jnp.float32),
            pltpu.SemaphoreType.DMA((2,)),
            pltpu.SemaphoreType.DMA((2,)),
        ],
    )
    def k(idx_hbm, table_hbm, out_hbm, idx_v, rows_v, gsem, wsem):
        wid = lax.axis_index("s") * NUM_CORES + lax.axis_index("c")
        base = wid * (n * CHUNK)
        pltpu.sync_copy(idx_hbm.at[wid], idx_v)

        def gather(j, b):
            return pltpu.make_async_copy(
                table_hbm.at[idx_v.at[j]], rows_v.at[b], gsem.at[b]
            )

        def writeback(j, b):
            return pltpu.make_async_copy(
                rows_v.at[b], out_hbm.at[pl.ds(base + j * CHUNK, CHUNK)], wsem.at[b]
            )

        gather(0, 0).start()

        @pl.loop(0, n, step=2)
        def chunk_loop(j0):
            for b in range(2):
                j = j0 + b
                gather(j, b).wait()

                @pl.when(j + 1 < n)
                def _start_next():
                    @pl.when(j >= 1)
                    def _free_buf():
                        writeback(j - 1, 1 - b).wait()

                    gather(j + 1, 1 - b).start()

                writeback(j, b).start()

        writeback(n - 2, 0).wait()
        writeback(n - 1, 1).wait()

    return k


def kernel(x, table):
    batch, hist = x.shape
    total = batch * hist
    chunks_per_worker = total // (NUM_WORKERS * CHUNK)
    idx = x.astype(jnp.int32).reshape(NUM_WORKERS, chunks_per_worker, CHUNK)
    out = _embed_kernel(total, chunks_per_worker)(idx, table)
    return out.reshape(batch, hist, EMBED_DIM)


# speedup vs baseline: 9.2056x; 1.2236x over previous
import functools

import jax
import jax.numpy as jnp
from jax import lax
from jax.experimental import pallas as pl
from jax.experimental.pallas import tpu as pltpu
from jax.experimental.pallas import tpu_sc as plsc

VOCAB = 100000
EMBED_DIM = 128
NUM_CORES = 2
NUM_SUBCORES = 16
NUM_WORKERS = NUM_CORES * NUM_SUBCORES
CHUNK = 128
NBUF = 4


def _embed_kernel(total, chunks_per_worker):
    mesh = plsc.VectorSubcoreMesh(core_axis_name="c", subcore_axis_name="s")

    n = chunks_per_worker

    @functools.partial(
        pl.kernel,
        out_type=jax.ShapeDtypeStruct((total, EMBED_DIM), jnp.float32),
        mesh=mesh,
        scratch_types=[
            pltpu.VMEM((n, CHUNK), jnp.int32),
            pltpu.VMEM((NBUF, CHUNK, EMBED_DIM), jnp.float32),
            pltpu.SemaphoreType.DMA((NBUF,)),
            pltpu.SemaphoreType.DMA((NBUF,)),
        ],
    )
    def k(idx_hbm, table_hbm, out_hbm, idx_v, rows_v, gsem, wsem):
        wid = lax.axis_index("s") * NUM_CORES + lax.axis_index("c")
        base = wid * (n * CHUNK)
        pltpu.sync_copy(idx_hbm.at[wid], idx_v)

        def gather(j, b):
            return pltpu.make_async_copy(
                table_hbm.at[idx_v.at[j]], rows_v.at[b], gsem.at[b]
            )

        def writeback(j, b):
            return pltpu.make_async_copy(
                rows_v.at[b], out_hbm.at[pl.ds(base + j * CHUNK, CHUNK)], wsem.at[b]
            )

        for jj in range(NBUF - 1):
            gather(jj, jj).start()

        @pl.loop(0, n, step=NBUF)
        def chunk_loop(j0):
            for b in range(NBUF):
                j = j0 + b
                gather(j, b).wait()
                writeback(j, b).start()
                nb = (b + NBUF - 1) % NBUF

                @pl.when(j + NBUF - 1 < n)
                def _start_next():
                    @pl.when(j >= 1)
                    def _free_buf():
                        writeback(j - 1, nb).wait()

                    gather(j + NBUF - 1, nb).start()

        for jj in range(n - NBUF, n):
            writeback(jj, jj % NBUF).wait()

    return k


def kernel(x, table):
    batch, hist = x.shape
    total = batch * hist
    chunks_per_worker = total // (NUM_WORKERS * CHUNK)
    idx = x.astype(jnp.int32).reshape(NUM_WORKERS, chunks_per_worker, CHUNK)
    out = _embed_kernel(total, chunks_per_worker)(idx, table)
    return out.reshape(batch, hist, EMBED_DIM)
